# Initial kernel scaffold; baseline (speedup 1.0000x reference)
import jax
import jax.numpy as jnp
from jax.experimental import pallas as pl


def kernel(x, emb):
    raise NotImplementedError("write your pallas kernel here")


# speedup vs baseline: 2.5490x; 2.5490x over previous
import jax
import jax.numpy as jnp
from jax.experimental import pallas as pl
from jax.experimental.pallas import tpu as pltpu

MAX_LEN = 8192
BS = 512


def _posemb_kernel(x_ref, emb_ref, bnd_ref, out_ref):
    em = emb_ref[...]
    rolled = jnp.concatenate([em[1:], bnd_ref[0]], axis=0)
    out_ref[0] = x_ref[0] + rolled


def kernel(x, emb):
    B, S, D = x.shape
    nj = S // BS
    bnd = jnp.concatenate([emb[BS:S:BS], emb[0:1]], axis=0).reshape(nj, 1, D)
    return pl.pallas_call(
        _posemb_kernel,
        grid=(nj, B),
        in_specs=[
            pl.BlockSpec((1, BS, D), lambda j, b: (b, j, 0)),
            pl.BlockSpec((BS, D), lambda j, b: (j, 0)),
            pl.BlockSpec((1, 1, D), lambda j, b: (j, 0, 0)),
        ],
        out_specs=pl.BlockSpec((1, BS, D), lambda j, b: (b, j, 0)),
        out_shape=jax.ShapeDtypeStruct(x.shape, x.dtype),
        compiler_params=pltpu.CompilerParams(
            dimension_semantics=("arbitrary", "arbitrary"),
        ),
    )(x, emb, bnd)


# speedup vs baseline: 2.8389x; 1.1138x over previous
import jax
import jax.numpy as jnp
from jax.experimental import pallas as pl
from jax.experimental.pallas import tpu as pltpu

MAX_LEN = 8192
BS = 1024


def _posemb_kernel(x_ref, emb_ref, bnd_ref, out_ref):
    em = emb_ref[...]
    rolled = jnp.concatenate([em[1:], bnd_ref[0]], axis=0)
    out_ref[0] = x_ref[0] + rolled


def kernel(x, emb):
    B, S, D = x.shape
    nj = S // BS
    bnd = jnp.concatenate([emb[BS:S:BS], emb[0:1]], axis=0).reshape(nj, 1, D)
    return pl.pallas_call(
        _posemb_kernel,
        grid=(nj, B),
        in_specs=[
            pl.BlockSpec((1, BS, D), lambda j, b: (b, j, 0)),
            pl.BlockSpec((BS, D), lambda j, b: (j, 0)),
            pl.BlockSpec((1, 1, D), lambda j, b: (j, 0, 0)),
        ],
        out_specs=pl.BlockSpec((1, BS, D), lambda j, b: (b, j, 0)),
        out_shape=jax.ShapeDtypeStruct(x.shape, x.dtype),
        compiler_params=pltpu.CompilerParams(
            dimension_semantics=("arbitrary", "arbitrary"),
        ),
    )(x, emb, bnd)


# speedup vs baseline: 2.9672x; 1.0452x over previous
import jax
import jax.numpy as jnp
from jax.experimental import pallas as pl
from jax.experimental.pallas import tpu as pltpu

MAX_LEN = 8192
BS = 2048


def _posemb_kernel(x_ref, emb_ref, bnd_ref, out_ref):
    em = emb_ref[...]
    rolled = jnp.concatenate([em[1:], bnd_ref[0]], axis=0)
    out_ref[0] = x_ref[0] + rolled


def kernel(x, emb):
    B, S, D = x.shape
    nj = S // BS
    bnd = jnp.concatenate([emb[BS:S:BS], emb[0:1]], axis=0).reshape(nj, 1, D)
    return pl.pallas_call(
        _posemb_kernel,
        grid=(nj, B),
        in_specs=[
            pl.BlockSpec((1, BS, D), lambda j, b: (b, j, 0)),
            pl.BlockSpec((BS, D), lambda j, b: (j, 0)),
            pl.BlockSpec((1, 1, D), lambda j, b: (j, 0, 0)),
        ],
        out_specs=pl.BlockSpec((1, BS, D), lambda j, b: (b, j, 0)),
        out_shape=jax.ShapeDtypeStruct(x.shape, x.dtype),
        compiler_params=pltpu.CompilerParams(
            dimension_semantics=("arbitrary", "arbitrary"),
        ),
    )(x, emb, bnd)
